# Initial kernel scaffold; baseline (speedup 1.0000x reference)
import jax
import jax.numpy as jnp
from jax.experimental import pallas as pl


def kernel(rois, cls_score):
    raise NotImplementedError("write your pallas kernel here")


# speedup vs baseline: 21.9599x; 21.9599x over previous
import functools

import jax
import jax.numpy as jnp
from jax import lax
from jax.experimental import pallas as pl
from jax.experimental.pallas import tpu as pltpu

_N = 20000
_ROWS = 160
_LANES = 128
_NP = _ROWS * _LANES
_MAX_OUT = 300
_OUT_ROWS = 304
_IOU_THR = 0.3
_NEG_INF = float("-inf")
_BIG = 2**30


def _nms_body(y1_ref, x1_ref, y2_ref, x2_ref, sc_in_ref, out_ref,
              ymin_ref, ymax_ref, xmin_ref, xmax_ref, area_ref, sc_ref,
              iota_ref):
    ymin = jnp.minimum(y1_ref[...], y2_ref[...])
    ymax = jnp.maximum(y1_ref[...], y2_ref[...])
    xmin = jnp.minimum(x1_ref[...], x2_ref[...])
    xmax = jnp.maximum(x1_ref[...], x2_ref[...])
    ymin_ref[...] = ymin
    ymax_ref[...] = ymax
    xmin_ref[...] = xmin
    xmax_ref[...] = xmax
    area_ref[...] = (ymax - ymin) * (xmax - xmin)
    sc_ref[...] = sc_in_ref[...]
    row = lax.broadcasted_iota(jnp.int32, (_ROWS, _LANES), 0)
    col = lax.broadcasted_iota(jnp.int32, (_ROWS, _LANES), 1)
    iota_ref[...] = row * _LANES + col
    lane_iota = col[0:1, :]

    def body(i, _):
        sc = sc_ref[...]
        m = jnp.max(sc)
        flat = iota_ref[...]
        idx = jnp.min(jnp.where(sc == m, flat, _BIG))
        valid = m > _NEG_INF
        out_ref[pl.ds(i, 1), :] = jnp.full(
            (1, _LANES), jnp.where(valid, idx, -1), dtype=jnp.int32)

        r = idx // _LANES
        c = idx % _LANES
        cmask = lane_iota == c

        def pick(ref):
            rowv = ref[pl.ds(r, 1), :]
            return jnp.sum(jnp.where(cmask, rowv, 0.0))

        by1 = pick(ymin_ref)
        by2 = pick(ymax_ref)
        bx1 = pick(xmin_ref)
        bx2 = pick(xmax_ref)
        ba = pick(area_ref)

        yy1 = jnp.maximum(ymin_ref[...], by1)
        xx1 = jnp.maximum(xmin_ref[...], bx1)
        yy2 = jnp.minimum(ymax_ref[...], by2)
        xx2 = jnp.minimum(xmax_ref[...], bx2)
        inter = jnp.maximum(yy2 - yy1, 0.0) * jnp.maximum(xx2 - xx1, 0.0)
        area = area_ref[...]
        denom = ba + area - inter
        iou = jnp.where((ba <= 0.0) | (area <= 0.0), 0.0, inter / denom)
        suppress = (iou > _IOU_THR) | (flat == idx)
        sc_ref[...] = jnp.where(suppress, _NEG_INF, sc)
        return 0

    lax.fori_loop(0, _MAX_OUT, body, 0)


@jax.jit
def kernel(rois, cls_score):
    def prep(v, pad_val):
        v = jnp.pad(v, (0, _NP - _N), constant_values=pad_val)
        return v.reshape(_ROWS, _LANES)

    y1 = prep(rois[:, 1], 0.0)
    x1 = prep(rois[:, 2], 0.0)
    y2 = prep(rois[:, 3], 0.0)
    x2 = prep(rois[:, 4], 0.0)
    sc = prep(jnp.reshape(cls_score, (-1,)), _NEG_INF)

    f32 = jnp.float32
    out = pl.pallas_call(
        _nms_body,
        out_shape=jax.ShapeDtypeStruct((_OUT_ROWS, _LANES), jnp.int32),
        in_specs=[pl.BlockSpec(memory_space=pltpu.VMEM)] * 5,
        out_specs=pl.BlockSpec(memory_space=pltpu.VMEM),
        scratch_shapes=[pltpu.VMEM((_ROWS, _LANES), f32)] * 6
        + [pltpu.VMEM((_ROWS, _LANES), jnp.int32)],
    )(y1, x1, y2, x2, sc)
    return out[:_MAX_OUT, 0]
